# baseline (speedup 1.0000x reference)
import functools

import jax
import jax.numpy as jnp
from jax import lax
from jax.experimental import pallas as pl
from jax.experimental.pallas import tpu as pltpu
from jax.experimental.pallas import tpu_sc as plsc

N = 10000
F = 64
H = 128
NC = 2
NS = 16
NW = NC * NS
CHUNK = 128
E = 320000
CPT = 80
EPT = CPT * CHUNK
EPAD = NW * EPT
NPAD = 10240
RPT = NPAD // NS
WB = RPT // CHUNK
NBUF = 2
PKM = 16383

_mesh = plsc.VectorSubcoreMesh(core_axis_name="c", subcore_axis_name="s")


@functools.partial(
    pl.kernel,
    mesh=_mesh,
    out_type=jax.ShapeDtypeStruct((NC * NPAD,), jnp.float32),
    scratch_types=[
        pltpu.VMEM((CPT, CHUNK), jnp.int32),
        pltpu.VMEM((CHUNK,), jnp.float32),
        pltpu.VMEM((RPT,), jnp.float32),
        pltpu.VMEM_SHARED((NPAD,), jnp.float32),
    ],
)
def _sc_degree(dst_hbm, deg_hbm, dstv, ones_v, zbuf, acc):
    cid = lax.axis_index("c")
    sid = lax.axis_index("s")
    wid = sid * NC + cid
    pltpu.sync_copy(dst_hbm.at[pl.ds(wid * CPT, CPT)], dstv)
    for i in range(CHUNK // 16):
        ones_v[pl.ds(i * 16, 16)] = jnp.ones((16,), jnp.float32)

    def zbody(i, carry):
        zbuf[pl.ds(i * 16, 16)] = jnp.zeros((16,), jnp.float32)
        return carry

    lax.fori_loop(0, RPT // 16, zbody, 0)
    pltpu.sync_copy(zbuf, acc.at[pl.ds(sid * RPT, RPT)])
    plsc.subcore_barrier()

    def body(j, carry):
        pltpu.sync_copy(ones_v, acc.at[dstv.at[j]], add=True)
        return carry

    lax.fori_loop(0, CPT, body, 0)
    plsc.subcore_barrier()
    pltpu.sync_copy(acc.at[pl.ds(sid * RPT, RPT)], zbuf)
    pltpu.sync_copy(zbuf, deg_hbm.at[pl.ds(cid * NPAD + sid * RPT, RPT)])


@functools.partial(
    pl.kernel,
    mesh=_mesh,
    out_type=jax.ShapeDtypeStruct((NC, NPAD, H), jnp.float32),
    scratch_types=[
        pltpu.VMEM((CPT + 8, CHUNK), jnp.int32),
    ]
    + [pltpu.VMEM((CHUNK,), jnp.int32) for _ in range(2)]
    + [pltpu.VMEM((CHUNK,), jnp.int32) for _ in range(2)]
    + [pltpu.VMEM((CHUNK, H), jnp.float32) for _ in range(2)]
    + [pltpu.VMEM_SHARED((NPAD, H), jnp.float32)]
    + [pltpu.SemaphoreType.DMA for _ in range(4)],
)
def _sc_spmm(g_hbm, pk_hbm, out_hbm, pk, *rest):
    sidx = rest[:2]
    didx = rest[2:4]
    rows = rest[4:6]
    acc = rest[6]
    gsem = rest[7:9]
    ssem = rest[9:11]
    cid = lax.axis_index("c")
    sid = lax.axis_index("s")
    wid = sid * NC + cid
    pltpu.sync_copy(pk_hbm.at[pl.ds(wid * CPT, CPT)], pk.at[pl.ds(0, CPT)])
    pltpu.sync_copy(pk_hbm.at[pl.ds(wid * CPT + CPT, 8)], pk.at[pl.ds(CPT, 8)])

    def zbody(i, carry):
        for k in range(H // 16):
            rows[0][i, pl.ds(k * 16, 16)] = jnp.zeros((16,), jnp.float32)
        return carry

    lax.fori_loop(0, CHUNK, zbody, 0)
    for k in range(WB):
        pltpu.sync_copy(rows[0], acc.at[pl.ds(sid * RPT + k * CHUNK, CHUNK)])
    plsc.subcore_barrier()

    def unpack(j, b):
        for k in range(CHUNK // 16):
            pe = pk[j, pl.ds(k * 16, 16)]
            sidx[b][pl.ds(k * 16, 16)] = lax.bitwise_and(pe, PKM)
            didx[b][pl.ds(k * 16, 16)] = lax.shift_right_logical(pe, 14)

    def gather(j, b):
        pltpu.async_copy(g_hbm.at[sidx[b]], rows[b], gsem[b])

    def gwait(b):
        pltpu.make_async_copy(g_hbm.at[sidx[b]], rows[b], gsem[b]).wait()

    def scatter(b):
        pltpu.async_copy(rows[b], acc.at[didx[b]], ssem[b], add=True)

    def swait(b):
        pltpu.make_async_copy(rows[b], acc.at[didx[b]], ssem[b]).wait()

    unpack(0, 0)
    gather(0, 0)
    unpack(1, 1)
    gather(1, 1)
    gwait(0)
    scatter(0)
    gwait(1)
    scatter(1)
    swait(0)
    unpack(2, 0)
    gather(2, 0)

    def body(i, carry):
        gwait(0)
        scatter(0)
        swait(1)
        unpack2 = 2 * i + 1
        unpack(unpack2, 1)
        gather(unpack2, 1)
        gwait(1)
        scatter(1)
        swait(0)
        unpack(2 * i + 2, 0)
        gather(2 * i + 2, 0)
        return carry

    lax.fori_loop(1, CPT // 2, body, 0)
    swait(1)
    gwait(0)
    plsc.subcore_barrier()
    for k in range(WB):
        pltpu.sync_copy(acc.at[pl.ds(sid * RPT + k * CHUNK, CHUNK)], rows[0])
        pltpu.sync_copy(rows[0], out_hbm.at[cid, pl.ds(sid * RPT + k * CHUNK, CHUNK)])


def _tc_first_body(x_ref, xm_ref, w0a_ref, w0b_ref, degp_ref, g_ref, dinv_ref):
    deg = 1.0 + degp_ref[0, :N, :] + degp_ref[1, :N, :]
    dinv = lax.rsqrt(deg)
    z = (jnp.dot(x_ref[...], w0a_ref[...], preferred_element_type=jnp.float32)
         + jnp.dot(xm_ref[...], w0b_ref[...], preferred_element_type=jnp.float32))
    dinv_ref[...] = dinv
    g_ref[...] = dinv * z


_tc_first = pl.pallas_call(
    _tc_first_body,
    out_shape=(jax.ShapeDtypeStruct((N, H), jnp.float32),
               jax.ShapeDtypeStruct((N, 1), jnp.float32)),
)


def _tc_mid_body(p_ref, g_ref, dinv_ref, b_ref, w_ref, gout_ref):
    dinv = dinv_ref[...]
    agg = p_ref[0, :N, :] + p_ref[1, :N, :] + g_ref[...]
    h = jnp.maximum(dinv * agg + b_ref[...], 0.0)
    gout_ref[...] = dinv * jnp.dot(h, w_ref[...],
                                   preferred_element_type=jnp.float32)


_tc_mid = pl.pallas_call(
    _tc_mid_body,
    out_shape=jax.ShapeDtypeStruct((N, H), jnp.float32),
)


def _tc_last_body(p_ref, g_ref, dinv_ref, b_ref, wr1_ref, br1_ref,
                  wr2r_ref, br2_ref, emb_ref, pred_ref):
    dinv = dinv_ref[...]
    emb = dinv * (p_ref[0, :N, :] + p_ref[1, :N, :] + g_ref[...]) + b_ref[...]
    emb_ref[...] = emb
    h = jnp.maximum(emb, 0.0)
    t = jnp.maximum(jnp.dot(h, wr1_ref[...], preferred_element_type=jnp.float32)
                    + br1_ref[...], 0.0)
    pred_ref[...] = jnp.sum(t * wr2r_ref[...], axis=1, keepdims=True) + br2_ref[...]


_tc_last = pl.pallas_call(
    _tc_last_body,
    out_shape=(jax.ShapeDtypeStruct((N, H), jnp.float32),
               jax.ShapeDtypeStruct((N, 1), jnp.float32)),
)


def kernel(x, x_mask, edge_index, W0, b0, W1, b1, W2, b2, W3, b3,
           Wr1, br1, Wr2, br2):
    src = edge_index[0].astype(jnp.int32)
    dst = edge_index[1].astype(jnp.int32)
    pad = EPAD - E + 8 * CHUNK
    ar = jnp.arange(pad, dtype=jnp.int32)
    src_p = jnp.concatenate([src, (ar * 97) % N]).reshape(NW * CPT + 8, CHUNK)
    dst_p = jnp.concatenate([dst, N + (ar % (NPAD - N))]).reshape(NW * CPT + 8, CHUNK)
    pk_p = dst_p * (PKM + 1) + src_p

    degp = _sc_degree(dst_p).reshape(NC, NPAD, 1)
    g0, dinv = _tc_first(x[:, :F], x_mask[:, :F], W0[:F], W0[F:], degp)
    p = _sc_spmm(g0, pk_p)
    g1 = _tc_mid(p, g0, dinv, b0.reshape(1, H), W1)
    p = _sc_spmm(g1, pk_p)
    g2 = _tc_mid(p, g1, dinv, b1.reshape(1, H), W2)
    p = _sc_spmm(g2, pk_p)
    g3 = _tc_mid(p, g2, dinv, b2.reshape(1, H), W3)
    p = _sc_spmm(g3, pk_p)
    emb, pred = _tc_last(p, g3, dinv, b3.reshape(1, H), Wr1,
                         br1.reshape(1, H), Wr2.reshape(1, H),
                         br2.reshape(1, 1))
    return emb, pred


# speedup vs baseline: 1.0764x; 1.0764x over previous
import functools

import jax
import jax.numpy as jnp
from jax import lax
from jax.experimental import pallas as pl
from jax.experimental.pallas import tpu as pltpu
from jax.experimental.pallas import tpu_sc as plsc

N = 10000
F = 64
H = 128
NC = 2
NS = 16
NW = NC * NS
CHUNK = 128
E = 320000
CPT = 80
EPT = CPT * CHUNK
EPAD = NW * EPT
NPAD = 10240
RPT = NPAD // NS
WB = RPT // CHUNK
SCH = 64
NCH = EPT // SCH
PKM = 16383

_mesh = plsc.VectorSubcoreMesh(core_axis_name="c", subcore_axis_name="s")


@functools.partial(
    pl.kernel,
    mesh=_mesh,
    out_type=jax.ShapeDtypeStruct((NC * NPAD,), jnp.float32),
    scratch_types=[
        pltpu.VMEM((CPT, CHUNK), jnp.int32),
        pltpu.VMEM((CHUNK,), jnp.float32),
        pltpu.VMEM((RPT,), jnp.float32),
        pltpu.VMEM_SHARED((NPAD,), jnp.float32),
    ],
)
def _sc_degree(dst_hbm, deg_hbm, dstv, ones_v, zbuf, acc):
    cid = lax.axis_index("c")
    sid = lax.axis_index("s")
    wid = sid * NC + cid
    pltpu.sync_copy(dst_hbm.at[pl.ds(wid * CPT, CPT)], dstv)
    for i in range(CHUNK // 16):
        ones_v[pl.ds(i * 16, 16)] = jnp.ones((16,), jnp.float32)

    def zbody(i, carry):
        zbuf[pl.ds(i * 16, 16)] = jnp.zeros((16,), jnp.float32)
        return carry

    lax.fori_loop(0, RPT // 16, zbody, 0)
    pltpu.sync_copy(zbuf, acc.at[pl.ds(sid * RPT, RPT)])
    plsc.subcore_barrier()

    def body(j, carry):
        pltpu.sync_copy(ones_v, acc.at[dstv.at[j]], add=True)
        return carry

    lax.fori_loop(0, CPT, body, 0)
    plsc.subcore_barrier()
    pltpu.sync_copy(acc.at[pl.ds(sid * RPT, RPT)], zbuf)
    pltpu.sync_copy(zbuf, deg_hbm.at[pl.ds(cid * NPAD + sid * RPT, RPT)])


@functools.partial(
    pl.kernel,
    mesh=_mesh,
    out_type=jax.ShapeDtypeStruct((NC, NPAD, H), jnp.float32),
    scratch_types=[
        pltpu.VMEM((CPT, CHUNK), jnp.int32),
    ]
    + [pltpu.VMEM((SCH,), jnp.int32) for _ in range(4)]
    + [pltpu.VMEM((SCH,), jnp.int32) for _ in range(4)]
    + [pltpu.VMEM((SCH, H), jnp.float32) for _ in range(4)]
    + [pltpu.VMEM_SHARED((NPAD, H), jnp.float32)]
    + [pltpu.SemaphoreType.DMA for _ in range(8)],
)
def _sc_spmm(g_hbm, pk_hbm, out_hbm, pk, *rest):
    sidx = rest[:4]
    didx = rest[4:8]
    rows = rest[8:12]
    acc = rest[12]
    gsem = rest[13:17]
    ssem = rest[17:21]
    cid = lax.axis_index("c")
    sid = lax.axis_index("s")
    wid = sid * NC + cid
    pltpu.sync_copy(pk_hbm.at[pl.ds(wid * CPT, CPT)], pk.at[pl.ds(0, CPT)])

    def zbody(i, carry):
        for k in range(H // 16):
            rows[0][i, pl.ds(k * 16, 16)] = jnp.zeros((16,), jnp.float32)
        return carry

    lax.fori_loop(0, SCH, zbody, 0)
    for k in range(RPT // SCH):
        pltpu.sync_copy(rows[0], acc.at[pl.ds(sid * RPT + k * SCH, SCH)])
    plsc.subcore_barrier()

    def unpack(row, colbase, slot):
        for k in range(SCH // 16):
            pe = pk[row, pl.ds(colbase + k * 16, 16)]
            sidx[slot][pl.ds(k * 16, 16)] = lax.bitwise_and(pe, PKM)
            didx[slot][pl.ds(k * 16, 16)] = lax.shift_right_logical(pe, 14)

    def gather(s):
        pltpu.async_copy(g_hbm.at[sidx[s]], rows[s], gsem[s])

    def gwait(s):
        pltpu.make_async_copy(g_hbm.at[sidx[s]], rows[s], gsem[s]).wait()

    def scatter(s):
        pltpu.async_copy(rows[s], acc.at[didx[s]], ssem[s], add=True)

    def swait(s):
        pltpu.make_async_copy(rows[s], acc.at[didx[s]], ssem[s]).wait()

    unpack(0, 0, 0)
    gather(0)
    unpack(0, SCH, 1)
    gather(1)
    gwait(0)
    scatter(0)
    unpack(1, 0, 2)
    gather(2)
    gwait(1)
    scatter(1)
    unpack(1, SCH, 3)
    gather(3)

    def body(i, carry):
        for b in range(4):
            s = (2 + b) % 4
            gwait(s)
            scatter(s)
            swait(b)
            unpack(2 * i + 2 + (b // 2), (b % 2) * SCH, b)
            gather(b)
        return carry

    lax.fori_loop(0, NCH // 4 - 1, body, 0)
    gwait(2)
    scatter(2)
    gwait(3)
    scatter(3)
    for s in range(4):
        swait(s)
    plsc.subcore_barrier()
    for k in range(RPT // SCH):
        pltpu.sync_copy(acc.at[pl.ds(sid * RPT + k * SCH, SCH)], rows[0])
        pltpu.sync_copy(rows[0], out_hbm.at[cid, pl.ds(sid * RPT + k * SCH, SCH)])


def _tc_first_body(x_ref, xm_ref, w0a_ref, w0b_ref, degp_ref, g_ref, dinv_ref):
    deg = 1.0 + degp_ref[0, :N, :] + degp_ref[1, :N, :]
    dinv = lax.rsqrt(deg)
    z = (jnp.dot(x_ref[...], w0a_ref[...], preferred_element_type=jnp.float32)
         + jnp.dot(xm_ref[...], w0b_ref[...], preferred_element_type=jnp.float32))
    dinv_ref[...] = dinv
    g_ref[...] = dinv * z


_tc_first = pl.pallas_call(
    _tc_first_body,
    out_shape=(jax.ShapeDtypeStruct((N, H), jnp.float32),
               jax.ShapeDtypeStruct((N, 1), jnp.float32)),
)


def _tc_mid_body(p_ref, g_ref, dinv_ref, b_ref, w_ref, gout_ref):
    dinv = dinv_ref[...]
    agg = p_ref[0, :N, :] + p_ref[1, :N, :] + g_ref[...]
    h = jnp.maximum(dinv * agg + b_ref[...], 0.0)
    gout_ref[...] = dinv * jnp.dot(h, w_ref[...],
                                   preferred_element_type=jnp.float32)


_tc_mid = pl.pallas_call(
    _tc_mid_body,
    out_shape=jax.ShapeDtypeStruct((N, H), jnp.float32),
)


def _tc_last_body(p_ref, g_ref, dinv_ref, b_ref, wr1_ref, br1_ref,
                  wr2r_ref, br2_ref, emb_ref, pred_ref):
    dinv = dinv_ref[...]
    emb = dinv * (p_ref[0, :N, :] + p_ref[1, :N, :] + g_ref[...]) + b_ref[...]
    emb_ref[...] = emb
    h = jnp.maximum(emb, 0.0)
    t = jnp.maximum(jnp.dot(h, wr1_ref[...], preferred_element_type=jnp.float32)
                    + br1_ref[...], 0.0)
    pred_ref[...] = jnp.sum(t * wr2r_ref[...], axis=1, keepdims=True) + br2_ref[...]


_tc_last = pl.pallas_call(
    _tc_last_body,
    out_shape=(jax.ShapeDtypeStruct((N, H), jnp.float32),
               jax.ShapeDtypeStruct((N, 1), jnp.float32)),
)


def kernel(x, x_mask, edge_index, W0, b0, W1, b1, W2, b2, W3, b3,
           Wr1, br1, Wr2, br2):
    src = edge_index[0].astype(jnp.int32)
    dst = edge_index[1].astype(jnp.int32)
    pad = EPAD - E + 8 * CHUNK
    ar = jnp.arange(pad, dtype=jnp.int32)
    src_p = jnp.concatenate([src, (ar * 97) % N]).reshape(NW * CPT + 8, CHUNK)
    dst_p = jnp.concatenate([dst, N + (ar % (NPAD - N))]).reshape(NW * CPT + 8, CHUNK)
    pk_p = dst_p * (PKM + 1) + src_p

    degp = _sc_degree(dst_p).reshape(NC, NPAD, 1)
    g0, dinv = _tc_first(x[:, :F], x_mask[:, :F], W0[:F], W0[F:], degp)
    p = _sc_spmm(g0, pk_p)
    g1 = _tc_mid(p, g0, dinv, b0.reshape(1, H), W1)
    p = _sc_spmm(g1, pk_p)
    g2 = _tc_mid(p, g1, dinv, b1.reshape(1, H), W2)
    p = _sc_spmm(g2, pk_p)
    g3 = _tc_mid(p, g2, dinv, b2.reshape(1, H), W3)
    p = _sc_spmm(g3, pk_p)
    emb, pred = _tc_last(p, g3, dinv, b3.reshape(1, H), Wr1,
                         br1.reshape(1, H), Wr2.reshape(1, H),
                         br2.reshape(1, 1))
    return emb, pred


# speedup vs baseline: 1.2153x; 1.1290x over previous
import functools

import jax
import jax.numpy as jnp
from jax import lax
from jax.experimental import pallas as pl
from jax.experimental.pallas import tpu as pltpu
from jax.experimental.pallas import tpu_sc as plsc

N = 10000
F = 64
H = 128
NC = 2
NS = 16
NW = NC * NS
CHUNK = 128
E = 320000
CPT = 80
EPT = CPT * CHUNK
EPAD = NW * EPT
NPAD = 10240
RPT = NPAD // NS
WB = RPT // CHUNK
SCH = 64
NCH = EPT // SCH
PKM = 16383

_mesh = plsc.VectorSubcoreMesh(core_axis_name="c", subcore_axis_name="s")


@functools.partial(
    pl.kernel,
    mesh=_mesh,
    out_type=jax.ShapeDtypeStruct((NC * NPAD,), jnp.float32),
    scratch_types=[
        pltpu.VMEM((CPT, CHUNK), jnp.int32),
        pltpu.VMEM((CHUNK,), jnp.float32),
        pltpu.VMEM((RPT,), jnp.float32),
        pltpu.VMEM_SHARED((NPAD,), jnp.float32),
    ],
)
def _sc_degree(dst_hbm, deg_hbm, dstv, ones_v, zbuf, acc):
    cid = lax.axis_index("c")
    sid = lax.axis_index("s")
    wid = sid * NC + cid
    pltpu.sync_copy(dst_hbm.at[pl.ds(wid * CPT, CPT)], dstv)
    for i in range(CHUNK // 16):
        ones_v[pl.ds(i * 16, 16)] = jnp.ones((16,), jnp.float32)

    def zbody(i, carry):
        zbuf[pl.ds(i * 16, 16)] = jnp.zeros((16,), jnp.float32)
        return carry

    lax.fori_loop(0, RPT // 16, zbody, 0)
    pltpu.sync_copy(zbuf, acc.at[pl.ds(sid * RPT, RPT)])
    plsc.subcore_barrier()

    def body(j, carry):
        pltpu.sync_copy(ones_v, acc.at[dstv.at[j]], add=True)
        return carry

    lax.fori_loop(0, CPT, body, 0)
    plsc.subcore_barrier()
    pltpu.sync_copy(acc.at[pl.ds(sid * RPT, RPT)], zbuf)
    pltpu.sync_copy(zbuf, deg_hbm.at[pl.ds(cid * NPAD + sid * RPT, RPT)])


@functools.partial(
    pl.kernel,
    mesh=_mesh,
    out_type=jax.ShapeDtypeStruct((NC, NPAD, H), jnp.float32),
    scratch_types=[
        pltpu.VMEM((CPT, CHUNK), jnp.int32),
    ]
    + [pltpu.VMEM((CHUNK,), jnp.int32) for _ in range(2)]
    + [pltpu.VMEM((CHUNK,), jnp.int32) for _ in range(2)]
    + [pltpu.VMEM((CHUNK, H), jnp.float32) for _ in range(2)]
    + [pltpu.VMEM_SHARED((NPAD, H), jnp.float32)]
    + [pltpu.SemaphoreType.DMA for _ in range(2)],
)
def _sc_spmm(g_hbm, pk_hbm, out_hbm, pk, *rest):
    sidx = rest[:2]
    didx = rest[2:4]
    rows = rest[4:6]
    acc = rest[6]
    sems = rest[7:9]
    cid = lax.axis_index("c")
    sid = lax.axis_index("s")
    wid = sid * NC + cid
    pltpu.sync_copy(pk_hbm.at[pl.ds(wid * CPT, CPT)], pk)

    def zbody(i, carry):
        for k in range(H // 16):
            rows[0][i, pl.ds(k * 16, 16)] = jnp.zeros((16,), jnp.float32)
        return carry

    lax.fori_loop(0, CHUNK, zbody, 0)
    for k in range(WB):
        pltpu.async_copy(rows[0], acc.at[pl.ds(sid * RPT + k * CHUNK, CHUNK)],
                         sems[0])
    for k in range(WB):
        pltpu.make_async_copy(rows[0], acc.at[pl.ds(sid * RPT, CHUNK)],
                              sems[0]).wait()
    plsc.subcore_barrier()

    def unpack(j, b):
        for k in range(CHUNK // 16):
            pe = pk[j, pl.ds(k * 16, 16)]
            sidx[b][pl.ds(k * 16, 16)] = lax.bitwise_and(pe, PKM)
            didx[b][pl.ds(k * 16, 16)] = lax.shift_right_logical(pe, 14)

    for b in range(2):
        unpack(b, b)
        pltpu.async_copy(g_hbm.at[sidx[b]], rows[b], sems[b])

    def body(i, carry):
        for b in range(2):
            j = i * 2 + b
            pltpu.make_async_copy(g_hbm.at[sidx[b]], rows[b], sems[b]).wait()
            pltpu.sync_copy(rows[b], acc.at[didx[b]], add=True)
            unpack(j + 2, b)
            pltpu.async_copy(g_hbm.at[sidx[b]], rows[b], sems[b])
        return carry

    lax.fori_loop(0, CPT // 2 - 1, body, 0)
    for b in range(2):
        pltpu.make_async_copy(g_hbm.at[sidx[b]], rows[b], sems[b]).wait()
        pltpu.sync_copy(rows[b], acc.at[didx[b]], add=True)
    plsc.subcore_barrier()
    pltpu.sync_copy(acc.at[pl.ds(sid * RPT, CHUNK)], rows[0])
    for k in range(WB):
        pltpu.async_copy(rows[k % 2],
                         out_hbm.at[cid, pl.ds(sid * RPT + k * CHUNK, CHUNK)],
                         sems[k % 2])
        if k + 1 < WB:
            pltpu.sync_copy(acc.at[pl.ds(sid * RPT + (k + 1) * CHUNK, CHUNK)],
                            rows[(k + 1) % 2])
        pltpu.make_async_copy(
            rows[k % 2],
            out_hbm.at[cid, pl.ds(sid * RPT + k * CHUNK, CHUNK)],
            sems[k % 2]).wait()


def _tc_first_body(x_ref, xm_ref, w0a_ref, w0b_ref, degp_ref, g_ref, dinv_ref):
    deg = 1.0 + degp_ref[0, :N, :] + degp_ref[1, :N, :]
    dinv = lax.rsqrt(deg)
    z = (jnp.dot(x_ref[...], w0a_ref[...], preferred_element_type=jnp.float32)
         + jnp.dot(xm_ref[...], w0b_ref[...], preferred_element_type=jnp.float32))
    dinv_ref[...] = dinv
    g_ref[...] = dinv * z


_tc_first = pl.pallas_call(
    _tc_first_body,
    out_shape=(jax.ShapeDtypeStruct((N, H), jnp.float32),
               jax.ShapeDtypeStruct((N, 1), jnp.float32)),
)


def _tc_mid_body(p_ref, g_ref, dinv_ref, b_ref, w_ref, gout_ref):
    dinv = dinv_ref[...]
    agg = p_ref[0, :N, :] + p_ref[1, :N, :] + g_ref[...]
    h = jnp.maximum(dinv * agg + b_ref[...], 0.0)
    gout_ref[...] = dinv * jnp.dot(h, w_ref[...],
                                   preferred_element_type=jnp.float32)


_tc_mid = pl.pallas_call(
    _tc_mid_body,
    out_shape=jax.ShapeDtypeStruct((N, H), jnp.float32),
)


def _tc_last_body(p_ref, g_ref, dinv_ref, b_ref, wr1_ref, br1_ref,
                  wr2r_ref, br2_ref, emb_ref, pred_ref):
    dinv = dinv_ref[...]
    emb = dinv * (p_ref[0, :N, :] + p_ref[1, :N, :] + g_ref[...]) + b_ref[...]
    emb_ref[...] = emb
    h = jnp.maximum(emb, 0.0)
    t = jnp.maximum(jnp.dot(h, wr1_ref[...], preferred_element_type=jnp.float32)
                    + br1_ref[...], 0.0)
    pred_ref[...] = jnp.sum(t * wr2r_ref[...], axis=1, keepdims=True) + br2_ref[...]


_tc_last = pl.pallas_call(
    _tc_last_body,
    out_shape=(jax.ShapeDtypeStruct((N, H), jnp.float32),
               jax.ShapeDtypeStruct((N, 1), jnp.float32)),
)


def kernel(x, x_mask, edge_index, W0, b0, W1, b1, W2, b2, W3, b3,
           Wr1, br1, Wr2, br2):
    src = edge_index[0].astype(jnp.int32)
    dst = edge_index[1].astype(jnp.int32)
    pad = EPAD - E + 8 * CHUNK
    ar = jnp.arange(pad, dtype=jnp.int32)
    src_p = jnp.concatenate([src, (ar * 97) % N]).reshape(NW * CPT + 8, CHUNK)
    dst_p = jnp.concatenate([dst, N + (ar % (NPAD - N))]).reshape(NW * CPT + 8, CHUNK)
    pk_p = dst_p * (PKM + 1) + src_p

    degp = _sc_degree(dst_p).reshape(NC, NPAD, 1)
    g0, dinv = _tc_first(x[:, :F], x_mask[:, :F], W0[:F], W0[F:], degp)
    p = _sc_spmm(g0, pk_p)
    g1 = _tc_mid(p, g0, dinv, b0.reshape(1, H), W1)
    p = _sc_spmm(g1, pk_p)
    g2 = _tc_mid(p, g1, dinv, b1.reshape(1, H), W2)
    p = _sc_spmm(g2, pk_p)
    g3 = _tc_mid(p, g2, dinv, b2.reshape(1, H), W3)
    p = _sc_spmm(g3, pk_p)
    emb, pred = _tc_last(p, g3, dinv, b3.reshape(1, H), Wr1,
                         br1.reshape(1, H), Wr2.reshape(1, H),
                         br2.reshape(1, 1))
    return emb, pred


# speedup vs baseline: 1.2332x; 1.0148x over previous
import functools

import jax
import jax.numpy as jnp
from jax import lax
from jax.experimental import pallas as pl
from jax.experimental.pallas import tpu as pltpu
from jax.experimental.pallas import tpu_sc as plsc

N = 10000
F = 64
H = 128
NC = 2
NS = 16
NW = NC * NS
CHUNK = 128
E = 320000
CPT = 80
EPT = CPT * CHUNK
EPAD = NW * EPT
NPAD = 10112
RPT = NPAD // NS
ZROW = 640
WBOFF = (0, 128, 256, 384, 512)
WBSZ = (128, 128, 128, 128, 120)

_mesh = plsc.VectorSubcoreMesh(core_axis_name="c", subcore_axis_name="s")


@functools.partial(
    pl.kernel,
    mesh=_mesh,
    out_type=jax.ShapeDtypeStruct((NC * NPAD,), jnp.float32),
    scratch_types=[
        pltpu.VMEM((CPT, CHUNK), jnp.int32),
        pltpu.VMEM((CHUNK,), jnp.float32),
        pltpu.VMEM((ZROW,), jnp.float32),
        pltpu.VMEM_SHARED((NPAD,), jnp.float32),
    ],
)
def _sc_degree(dst_hbm, deg_hbm, dstv, ones_v, zbuf, acc):
    cid = lax.axis_index("c")
    sid = lax.axis_index("s")
    wid = sid * NC + cid
    pltpu.sync_copy(dst_hbm.at[pl.ds(wid * CPT, CPT)], dstv)
    for i in range(CHUNK // 16):
        ones_v[pl.ds(i * 16, 16)] = jnp.ones((16,), jnp.float32)

    def zbody(i, carry):
        zbuf[pl.ds(i * 16, 16)] = jnp.zeros((16,), jnp.float32)
        return carry

    lax.fori_loop(0, ZROW // 16, zbody, 0)
    pltpu.sync_copy(zbuf.at[pl.ds(0, RPT)], acc.at[pl.ds(sid * RPT, RPT)])
    plsc.subcore_barrier()

    def body(j, carry):
        pltpu.sync_copy(ones_v, acc.at[dstv.at[j]], add=True)
        return carry

    lax.fori_loop(0, CPT, body, 0)
    plsc.subcore_barrier()
    pltpu.sync_copy(acc.at[pl.ds(sid * RPT, RPT)], zbuf.at[pl.ds(0, RPT)])
    pltpu.sync_copy(zbuf.at[pl.ds(0, RPT)],
                    deg_hbm.at[pl.ds(cid * NPAD + sid * RPT, RPT)])


@functools.partial(
    pl.kernel,
    mesh=_mesh,
    out_type=jax.ShapeDtypeStruct((NC, NPAD, H), jnp.float32),
    scratch_types=[pltpu.VMEM((CHUNK,), jnp.int32) for _ in range(3)]
    + [pltpu.VMEM((CHUNK,), jnp.int32) for _ in range(3)]
    + [pltpu.VMEM((CHUNK, H), jnp.float32) for _ in range(3)]
    + [pltpu.VMEM_SHARED((NPAD, H), jnp.float32)]
    + [pltpu.SemaphoreType.DMA for _ in range(6)],
)
def _sc_spmm(g_hbm, src_hbm, dst_hbm, out_hbm, *rest):
    srcv = rest[:3]
    dstv = rest[3:6]
    rows = rest[6:9]
    acc = rest[9]
    gsem = rest[10:13]
    isem = rest[13:16]
    cid = lax.axis_index("c")
    sid = lax.axis_index("s")
    wid = sid * NC + cid
    ibase = wid * CPT * CHUNK

    def iload(j, s):
        pltpu.async_copy(src_hbm.at[pl.ds(ibase + j * CHUNK, CHUNK)],
                         srcv[s], isem[s])
        pltpu.async_copy(dst_hbm.at[pl.ds(ibase + j * CHUNK, CHUNK)],
                         dstv[s], isem[s])

    def iwait(s):
        pltpu.make_async_copy(src_hbm.at[pl.ds(ibase, CHUNK)],
                              srcv[s], isem[s]).wait()
        pltpu.make_async_copy(dst_hbm.at[pl.ds(ibase, CHUNK)],
                              dstv[s], isem[s]).wait()

    def gissue(s):
        pltpu.async_copy(g_hbm.at[srcv[s]], rows[s], gsem[s])

    def gwait(s):
        pltpu.make_async_copy(g_hbm.at[srcv[s]], rows[s], gsem[s]).wait()

    def scat(s):
        pltpu.sync_copy(rows[s], acc.at[dstv[s]], add=True)

    iload(0, 0)
    iload(1, 1)
    iload(2, 2)

    def zbody(i, carry):
        for k in range(H // 16):
            rows[0][i, pl.ds(k * 16, 16)] = jnp.zeros((16,), jnp.float32)
        return carry

    lax.fori_loop(0, CHUNK, zbody, 0)
    for o, sz in zip(WBOFF, WBSZ):
        pltpu.async_copy(rows[0].at[pl.ds(0, sz)],
                         acc.at[pl.ds(sid * RPT + o, sz)], gsem[0])
    for o, sz in zip(WBOFF, WBSZ):
        pltpu.make_async_copy(rows[0].at[pl.ds(0, sz)],
                              acc.at[pl.ds(sid * RPT + o, sz)], gsem[0]).wait()
    plsc.subcore_barrier()

    iwait(0)
    gissue(0)
    iwait(1)
    gissue(1)

    def step(j, s, tail):
        gwait(s)
        scat(s)
        if not tail:
            iwait((s + 2) % 3)
            gissue((s + 2) % 3)
            iload(j + 3, s)

    def body(i, carry):
        for r in range(3):
            step(3 * i + r, r, False)
        return carry

    lax.fori_loop(0, (CPT - 2) // 3, body, 0)
    step(CPT - 2, (CPT - 2) % 3, True)
    step(CPT - 1, (CPT - 1) % 3, True)
    iwait(CPT % 3)
    plsc.subcore_barrier()
    pltpu.sync_copy(acc.at[pl.ds(sid * RPT + WBOFF[0], WBSZ[0])],
                    rows[0].at[pl.ds(0, WBSZ[0])])
    for k in range(len(WBOFF)):
        b = k % 2
        pltpu.async_copy(rows[b].at[pl.ds(0, WBSZ[k])],
                         out_hbm.at[cid, pl.ds(sid * RPT + WBOFF[k], WBSZ[k])],
                         gsem[b])
        if k + 1 < len(WBOFF):
            pltpu.sync_copy(
                acc.at[pl.ds(sid * RPT + WBOFF[k + 1], WBSZ[k + 1])],
                rows[1 - b].at[pl.ds(0, WBSZ[k + 1])])
        pltpu.make_async_copy(
            rows[b].at[pl.ds(0, WBSZ[k])],
            out_hbm.at[cid, pl.ds(sid * RPT + WBOFF[k], WBSZ[k])],
            gsem[b]).wait()


def _tc_first_body(x_ref, xm_ref, w0a_ref, w0b_ref, degp_ref, g_ref, dinv_ref):
    deg = 1.0 + degp_ref[0, :N, :] + degp_ref[1, :N, :]
    dinv = lax.rsqrt(deg)
    z = (jnp.dot(x_ref[...], w0a_ref[...], preferred_element_type=jnp.float32)
         + jnp.dot(xm_ref[...], w0b_ref[...], preferred_element_type=jnp.float32))
    dinv_ref[...] = dinv
    g_ref[...] = dinv * z


_tc_first = pl.pallas_call(
    _tc_first_body,
    out_shape=(jax.ShapeDtypeStruct((N, H), jnp.float32),
               jax.ShapeDtypeStruct((N, 1), jnp.float32)),
)


def _tc_mid_body(p_ref, g_ref, dinv_ref, b_ref, w_ref, gout_ref):
    dinv = dinv_ref[...]
    agg = p_ref[0, :N, :] + p_ref[1, :N, :] + g_ref[...]
    h = jnp.maximum(dinv * agg + b_ref[...], 0.0)
    gout_ref[...] = dinv * jnp.dot(h, w_ref[...],
                                   preferred_element_type=jnp.float32)


_tc_mid = pl.pallas_call(
    _tc_mid_body,
    out_shape=jax.ShapeDtypeStruct((N, H), jnp.float32),
)


def _tc_last_body(p_ref, g_ref, dinv_ref, b_ref, wr1_ref, br1_ref,
                  wr2r_ref, br2_ref, emb_ref, pred_ref):
    dinv = dinv_ref[...]
    emb = dinv * (p_ref[0, :N, :] + p_ref[1, :N, :] + g_ref[...]) + b_ref[...]
    emb_ref[...] = emb
    h = jnp.maximum(emb, 0.0)
    t = jnp.maximum(jnp.dot(h, wr1_ref[...], preferred_element_type=jnp.float32)
                    + br1_ref[...], 0.0)
    pred_ref[...] = jnp.sum(t * wr2r_ref[...], axis=1, keepdims=True) + br2_ref[...]


_tc_last = pl.pallas_call(
    _tc_last_body,
    out_shape=(jax.ShapeDtypeStruct((N, H), jnp.float32),
               jax.ShapeDtypeStruct((N, 1), jnp.float32)),
)


def kernel(x, x_mask, edge_index, W0, b0, W1, b1, W2, b2, W3, b3,
           Wr1, br1, Wr2, br2):
    src = edge_index[0].astype(jnp.int32)
    dst = edge_index[1].astype(jnp.int32)
    pad = EPAD - E + 8 * CHUNK
    ar = jnp.arange(pad, dtype=jnp.int32)
    src_p = jnp.concatenate([src, (ar * 97) % N]).reshape(NW * CPT + 8, CHUNK)
    dst_p = jnp.concatenate([dst, N + (ar % (NPAD - N))]).reshape(NW * CPT + 8, CHUNK)
    src1 = src_p.reshape(-1)
    dst1 = dst_p.reshape(-1)

    degp = _sc_degree(dst_p).reshape(NC, NPAD, 1)
    g0, dinv = _tc_first(x[:, :F], x_mask[:, :F], W0[:F], W0[F:], degp)
    p = _sc_spmm(g0, src1, dst1)
    g1 = _tc_mid(p, g0, dinv, b0.reshape(1, H), W1)
    p = _sc_spmm(g1, src1, dst1)
    g2 = _tc_mid(p, g1, dinv, b1.reshape(1, H), W2)
    p = _sc_spmm(g2, src1, dst1)
    g3 = _tc_mid(p, g2, dinv, b2.reshape(1, H), W3)
    p = _sc_spmm(g3, src1, dst1)
    emb, pred = _tc_last(p, g3, dinv, b3.reshape(1, H), Wr1,
                         br1.reshape(1, H), Wr2.reshape(1, H),
                         br2.reshape(1, 1))
    return emb, pred
